# baseline (speedup 1.0000x reference)
import jax
import jax.numpy as jnp
from jax.experimental import pallas as pl
from jax.experimental.pallas import tpu as pltpu

_C = 128
_H = 56
_W = 56
_HW = _H * _W
_KW = 9 * _C + 8


def _bb_kernel(x_ref, w1_ref, w2_ref, m_ref, br_ref, o_ref, p_ref, h_ref):
    x = x_ref[0]
    p_ref[9 * _C:, :] = br_ref[...]

    def build_patches(src):
        for ky in range(3):
            for kx in range(3):
                t = ky * 3 + kx
                s = (ky - 1) * _W + (kx - 1)
                r = pltpu.roll(src, (-s) % _HW, 1) if s else src
                rb = r.astype(jnp.bfloat16)
                if t != 4:
                    rb = rb * m_ref[t]
                p_ref[t * _C:(t + 1) * _C, :] = rb

    build_patches(x)
    h = jnp.dot(w1_ref[...], p_ref[...], preferred_element_type=jnp.float32)
    h_ref[...] = jnp.maximum(h, 0.0)

    build_patches(h_ref[...])
    y = jnp.dot(w2_ref[...], p_ref[...], preferred_element_type=jnp.float32)
    o_ref[0] = jnp.maximum(y + x, 0.0)


def _prep_weights(w_oihw, gamma, beta, mean, var, eps):
    inv = gamma / jnp.sqrt(var + eps)
    wm = jnp.transpose(w_oihw, (0, 2, 3, 1)).reshape(_C, 9 * _C)
    wm = wm * inv[:, None]
    bias = (beta - mean * inv)[:, None]
    pad = jnp.zeros((_C, 7), jnp.float32)
    return jnp.concatenate([wm, bias, pad], axis=1).astype(jnp.bfloat16)


def _masks():
    lane = jnp.arange(_HW, dtype=jnp.int32)
    yc, xc = lane // _W, lane % _W
    rows = []
    for ky in range(3):
        for kx in range(3):
            valid = ((yc + ky - 1 >= 0) & (yc + ky - 1 < _H)
                     & (xc + kx - 1 >= 0) & (xc + kx - 1 < _W))
            rows.append(valid)
    m = jnp.stack(rows).astype(jnp.bfloat16)
    return jnp.broadcast_to(m[:, None, :], (9, _C, _HW))


def kernel(x, w1, gamma1, beta1, mean1, var1, w2, gamma2, beta2, mean2, var2,
           eps=1e-5):
    N = x.shape[0]
    xr = x.reshape(N, _C, _HW).astype(jnp.float32)
    w1m = _prep_weights(w1, gamma1, beta1, mean1, var1, eps)
    w2m = _prep_weights(w2, gamma2, beta2, mean2, var2, eps)
    masks = _masks()
    bias_rows = jnp.zeros((8, _HW), jnp.bfloat16).at[0, :].set(1.0)

    flops = 4 * N * _HW * 9 * _C * _C
    bytes_accessed = 2 * N * _C * _HW * 4 + 2 * _C * _KW * 2

    out = pl.pallas_call(
        _bb_kernel,
        out_shape=jax.ShapeDtypeStruct((N, _C, _HW), jnp.float32),
        grid=(N,),
        in_specs=[
            pl.BlockSpec((1, _C, _HW), lambda n: (n, 0, 0)),
            pl.BlockSpec((_C, _KW), lambda n: (0, 0)),
            pl.BlockSpec((_C, _KW), lambda n: (0, 0)),
            pl.BlockSpec((9, _C, _HW), lambda n: (0, 0, 0)),
            pl.BlockSpec((8, _HW), lambda n: (0, 0)),
        ],
        out_specs=pl.BlockSpec((1, _C, _HW), lambda n: (n, 0, 0)),
        scratch_shapes=[
            pltpu.VMEM((_KW, _HW), jnp.bfloat16),
            pltpu.VMEM((_C, _HW), jnp.float32),
        ],
        compiler_params=pltpu.CompilerParams(
            dimension_semantics=("parallel",),
            vmem_limit_bytes=100 * 1024 * 1024,
        ),
        cost_estimate=pl.CostEstimate(flops=flops, transcendentals=0,
                                      bytes_accessed=bytes_accessed),
    )(xr, w1m, w2m, masks, bias_rows)

    return out.reshape(N, _C, _H, _W)


# speedup vs baseline: 1.7095x; 1.7095x over previous
import jax
import jax.numpy as jnp
from jax.experimental import pallas as pl
from jax.experimental.pallas import tpu as pltpu

_C = 128
_H = 56
_W = 56
_HW = _H * _W
_KW = 3 * _C + 8


def _roll(v, k):
    return jnp.concatenate([v[:, _HW - k:], v[:, :_HW - k]], axis=1)


def _bb_kernel(x_ref, w1_ref, w2_ref, myb_ref, mxf_ref, br_ref, o_ref,
               p_ref, h_ref):
    x = x_ref[0]
    p_ref[3 * _C:, :] = br_ref[...]

    def conv(src, w_ref):
        dn = _roll(src, _W)
        up = _roll(src, _HW - _W)
        p_ref[0:_C, :] = dn.astype(jnp.bfloat16) * myb_ref[0]
        p_ref[_C:2 * _C, :] = src.astype(jnp.bfloat16)
        p_ref[2 * _C:3 * _C, :] = up.astype(jnp.bfloat16) * myb_ref[1]
        p = p_ref[...]
        z0 = jnp.dot(w_ref[0], p, preferred_element_type=jnp.float32)
        z1 = jnp.dot(w_ref[1], p, preferred_element_type=jnp.float32)
        z2 = jnp.dot(w_ref[2], p, preferred_element_type=jnp.float32)
        return (_roll(z0, 1) * mxf_ref[0] + z1
                + _roll(z2, _HW - 1) * mxf_ref[1])

    h_ref[...] = jnp.maximum(conv(x, w1_ref), 0.0)
    o_ref[0] = jnp.maximum(conv(h_ref[...], w2_ref) + x, 0.0)


def _prep_weights(w_oihw, gamma, beta, mean, var, eps):
    inv = gamma / jnp.sqrt(var + eps)
    wm = jnp.transpose(w_oihw, (3, 0, 2, 1)).reshape(3, _C, 3 * _C)
    wm = wm * inv[None, :, None]
    bias = (beta - mean * inv)[:, None]
    ext = jnp.zeros((3, _C, 8), jnp.float32).at[1, :, 0:1].set(bias)
    return jnp.concatenate([wm, ext], axis=2).astype(jnp.bfloat16)


def _masks():
    lane = jnp.arange(_HW, dtype=jnp.int32)
    yc, xc = lane // _W, lane % _W
    my = jnp.stack([yc >= 1, yc < _H - 1])
    mx = jnp.stack([xc >= 1, xc < _W - 1])
    myb = jnp.broadcast_to(my[:, None, :], (2, _C, _HW)).astype(jnp.bfloat16)
    mxf = jnp.broadcast_to(mx[:, None, :], (2, _C, _HW)).astype(jnp.float32)
    return myb, mxf


def kernel(x, w1, gamma1, beta1, mean1, var1, w2, gamma2, beta2, mean2, var2,
           eps=1e-5):
    N = x.shape[0]
    xr = x.reshape(N, _C, _HW).astype(jnp.float32)
    w1m = _prep_weights(w1, gamma1, beta1, mean1, var1, eps)
    w2m = _prep_weights(w2, gamma2, beta2, mean2, var2, eps)
    myb, mxf = _masks()
    bias_rows = jnp.zeros((8, _HW), jnp.bfloat16).at[0, :].set(1.0)

    flops = 4 * N * _HW * 9 * _C * _C
    bytes_accessed = 2 * N * _C * _HW * 4 + 2 * 3 * _C * _KW * 2

    out = pl.pallas_call(
        _bb_kernel,
        out_shape=jax.ShapeDtypeStruct((N, _C, _HW), jnp.float32),
        grid=(N,),
        in_specs=[
            pl.BlockSpec((1, _C, _HW), lambda n: (n, 0, 0)),
            pl.BlockSpec((3, _C, _KW), lambda n: (0, 0, 0)),
            pl.BlockSpec((3, _C, _KW), lambda n: (0, 0, 0)),
            pl.BlockSpec((2, _C, _HW), lambda n: (0, 0, 0)),
            pl.BlockSpec((2, _C, _HW), lambda n: (0, 0, 0)),
            pl.BlockSpec((8, _HW), lambda n: (0, 0)),
        ],
        out_specs=pl.BlockSpec((1, _C, _HW), lambda n: (n, 0, 0)),
        scratch_shapes=[
            pltpu.VMEM((_KW, _HW), jnp.bfloat16),
            pltpu.VMEM((_C, _HW), jnp.float32),
        ],
        compiler_params=pltpu.CompilerParams(
            dimension_semantics=("parallel",),
            vmem_limit_bytes=100 * 1024 * 1024,
        ),
        cost_estimate=pl.CostEstimate(flops=flops, transcendentals=0,
                                      bytes_accessed=bytes_accessed),
    )(xr, w1m, w2m, myb, mxf, bias_rows)

    return out.reshape(N, _C, _H, _W)


# speedup vs baseline: 1.7873x; 1.0455x over previous
import jax
import jax.numpy as jnp
from jax.experimental import pallas as pl
from jax.experimental.pallas import tpu as pltpu

_C = 128
_H = 56
_W = 56
_HW = _H * _W
_KW = 3 * _C + 8


def _roll(v, k):
    return jnp.concatenate([v[:, _HW - k:], v[:, :_HW - k]], axis=1)


def _bb_kernel(x_ref, w1_ref, w2_ref, myb_ref, mxf_ref, br_ref, o_ref,
               p_ref, h_ref):
    x = x_ref[0]
    p_ref[3 * _C:, :] = br_ref[...]

    def conv(src, w_ref):
        dn = _roll(src, _W)
        up = _roll(src, _HW - _W)
        p_ref[0:_C, :] = dn.astype(jnp.bfloat16) * myb_ref[0]
        p_ref[_C:2 * _C, :] = src.astype(jnp.bfloat16)
        p_ref[2 * _C:3 * _C, :] = up.astype(jnp.bfloat16) * myb_ref[1]
        z = jnp.dot(w_ref[...], p_ref[...], preferred_element_type=jnp.float32)
        return (_roll(z[0:_C], 1) * mxf_ref[0] + z[_C:2 * _C]
                + _roll(z[2 * _C:3 * _C], _HW - 1) * mxf_ref[1])

    h_ref[...] = jnp.maximum(conv(x, w1_ref), 0.0)
    o_ref[0] = jnp.maximum(conv(h_ref[...], w2_ref) + x, 0.0)


def _prep_weights(w_oihw, gamma, beta, mean, var, eps):
    inv = gamma / jnp.sqrt(var + eps)
    wm = jnp.transpose(w_oihw, (3, 0, 2, 1)).reshape(3, _C, 3 * _C)
    wm = wm * inv[None, :, None]
    bias = (beta - mean * inv)[:, None]
    ext = jnp.zeros((3, _C, 8), jnp.float32).at[1, :, 0:1].set(bias)
    return jnp.concatenate([wm, ext], axis=2).reshape(3 * _C, _KW).astype(
        jnp.bfloat16)


def _masks():
    lane = jnp.arange(_HW, dtype=jnp.int32)
    yc, xc = lane // _W, lane % _W
    my = jnp.stack([yc >= 1, yc < _H - 1])
    mx = jnp.stack([xc >= 1, xc < _W - 1])
    myb = jnp.broadcast_to(my[:, None, :], (2, _C, _HW)).astype(jnp.bfloat16)
    mxf = jnp.broadcast_to(mx[:, None, :], (2, _C, _HW)).astype(jnp.float32)
    return myb, mxf


def kernel(x, w1, gamma1, beta1, mean1, var1, w2, gamma2, beta2, mean2, var2,
           eps=1e-5):
    N = x.shape[0]
    xr = x.reshape(N, _C, _HW).astype(jnp.float32)
    w1m = _prep_weights(w1, gamma1, beta1, mean1, var1, eps)
    w2m = _prep_weights(w2, gamma2, beta2, mean2, var2, eps)
    myb, mxf = _masks()
    bias_rows = jnp.zeros((8, _HW), jnp.bfloat16).at[0, :].set(1.0)

    flops = 4 * N * _HW * 9 * _C * _C
    bytes_accessed = 2 * N * _C * _HW * 4 + 2 * 3 * _C * _KW * 2

    out = pl.pallas_call(
        _bb_kernel,
        out_shape=jax.ShapeDtypeStruct((N, _C, _HW), jnp.float32),
        grid=(N,),
        in_specs=[
            pl.BlockSpec((1, _C, _HW), lambda n: (n, 0, 0)),
            pl.BlockSpec((3 * _C, _KW), lambda n: (0, 0)),
            pl.BlockSpec((3 * _C, _KW), lambda n: (0, 0)),
            pl.BlockSpec((2, _C, _HW), lambda n: (0, 0, 0)),
            pl.BlockSpec((2, _C, _HW), lambda n: (0, 0, 0)),
            pl.BlockSpec((8, _HW), lambda n: (0, 0)),
        ],
        out_specs=pl.BlockSpec((1, _C, _HW), lambda n: (n, 0, 0)),
        scratch_shapes=[
            pltpu.VMEM((_KW, _HW), jnp.bfloat16),
            pltpu.VMEM((_C, _HW), jnp.float32),
        ],
        compiler_params=pltpu.CompilerParams(
            dimension_semantics=("parallel",),
            vmem_limit_bytes=100 * 1024 * 1024,
        ),
        cost_estimate=pl.CostEstimate(flops=flops, transcendentals=0,
                                      bytes_accessed=bytes_accessed),
    )(xr, w1m, w2m, myb, mxf, bias_rows)

    return out.reshape(N, _C, _H, _W)


# speedup vs baseline: 1.8246x; 1.0208x over previous
import jax
import jax.numpy as jnp
from jax.experimental import pallas as pl
from jax.experimental.pallas import tpu as pltpu

_C = 128
_H = 56
_W = 56
_HW = _H * _W
_KW = 3 * _C + 8


def _roll(v, k):
    return jnp.concatenate([v[:, _HW - k:], v[:, :_HW - k]], axis=1)


def _bb_kernel(x_ref, w1_ref, w2_ref, myb_ref, mxf_ref, br_ref, o_ref,
               p_ref, h_ref):
    x = x_ref[0]
    p_ref[3 * _C:, :] = br_ref[...]

    def conv(src, w_ref):
        dn = _roll(src, _W)
        up = _roll(src, _HW - _W)
        p_ref[0:_C, :] = dn.astype(jnp.bfloat16) * myb_ref[0]
        p_ref[_C:2 * _C, :] = src.astype(jnp.bfloat16)
        p_ref[2 * _C:3 * _C, :] = up.astype(jnp.bfloat16) * myb_ref[1]
        z = jnp.dot(w_ref[...], p_ref[...], preferred_element_type=jnp.float32)
        return (_roll(z[0:_C], 1) * mxf_ref[0] + z[_C:2 * _C]
                + _roll(z[2 * _C:3 * _C], _HW - 1) * mxf_ref[1])

    h_ref[...] = jnp.maximum(conv(x, w1_ref), 0.0)
    o_ref[0] = jnp.maximum(conv(h_ref[...], w2_ref) + x, 0.0)


def _prep_weights(w_oihw, gamma, beta, mean, var, eps):
    inv = gamma / jnp.sqrt(var + eps)
    wm = jnp.transpose(w_oihw, (3, 0, 2, 1)).reshape(3, _C, 3 * _C)
    wm = wm * inv[None, :, None]
    bias = (beta - mean * inv)[:, None]
    ext = jnp.zeros((3, _C, 8), jnp.float32).at[1, :, 0:1].set(bias)
    return jnp.concatenate([wm, ext], axis=2).reshape(3 * _C, _KW).astype(
        jnp.bfloat16)


def _masks():
    lane = jnp.arange(_HW, dtype=jnp.int32)
    yc, xc = lane // _W, lane % _W
    my = jnp.stack([yc >= 1, yc < _H - 1])
    mx = jnp.stack([xc >= 1, xc < _W - 1])
    myb = my[:, None, :].astype(jnp.bfloat16)
    mxf = mx[:, None, :].astype(jnp.float32)
    return myb, mxf


def kernel(x, w1, gamma1, beta1, mean1, var1, w2, gamma2, beta2, mean2, var2,
           eps=1e-5):
    N = x.shape[0]
    xr = x.reshape(N, _C, _HW).astype(jnp.float32)
    w1m = _prep_weights(w1, gamma1, beta1, mean1, var1, eps)
    w2m = _prep_weights(w2, gamma2, beta2, mean2, var2, eps)
    myb, mxf = _masks()
    bias_rows = jnp.zeros((8, _HW), jnp.bfloat16).at[0, :].set(1.0)

    flops = 4 * N * _HW * 9 * _C * _C
    bytes_accessed = 2 * N * _C * _HW * 4 + 2 * 3 * _C * _KW * 2

    out = pl.pallas_call(
        _bb_kernel,
        out_shape=jax.ShapeDtypeStruct((N, _C, _HW), jnp.float32),
        grid=(N,),
        in_specs=[
            pl.BlockSpec((1, _C, _HW), lambda n: (n, 0, 0)),
            pl.BlockSpec((3 * _C, _KW), lambda n: (0, 0)),
            pl.BlockSpec((3 * _C, _KW), lambda n: (0, 0)),
            pl.BlockSpec((2, 1, _HW), lambda n: (0, 0, 0)),
            pl.BlockSpec((2, 1, _HW), lambda n: (0, 0, 0)),
            pl.BlockSpec((8, _HW), lambda n: (0, 0)),
        ],
        out_specs=pl.BlockSpec((1, _C, _HW), lambda n: (n, 0, 0)),
        scratch_shapes=[
            pltpu.VMEM((_KW, _HW), jnp.bfloat16),
            pltpu.VMEM((_C, _HW), jnp.float32),
        ],
        compiler_params=pltpu.CompilerParams(
            dimension_semantics=("parallel",),
            vmem_limit_bytes=100 * 1024 * 1024,
        ),
        cost_estimate=pl.CostEstimate(flops=flops, transcendentals=0,
                                      bytes_accessed=bytes_accessed),
    )(xr, w1m, w2m, myb, mxf, bias_rows)

    return out.reshape(N, _C, _H, _W)


# speedup vs baseline: 1.8411x; 1.0090x over previous
import jax
import jax.numpy as jnp
from jax.experimental import pallas as pl
from jax.experimental.pallas import tpu as pltpu

_C = 128
_H = 56
_W = 56
_HW = _H * _W
_KW = 3 * _C + 8
_IPS = 2


def _roll(v, k):
    return jnp.concatenate([v[:, _HW - k:], v[:, :_HW - k]], axis=1)


def _roll32(v, k):
    return jnp.concatenate([v[:, _HW - k:], v[:, :_HW - k]], axis=1)


def _bb_kernel(x_ref, w1_ref, w2_ref, myb_ref, mxf_ref, br_ref, o_ref,
               p_ref, h_ref):
    p_ref[3 * _C:, :] = br_ref[...]

    def conv(src, w_ref):
        sb = src.astype(jnp.bfloat16)
        sbi = pltpu.bitcast(sb, jnp.int32)
        dn = pltpu.bitcast(_roll32(sbi, _W), jnp.bfloat16)
        up = pltpu.bitcast(_roll32(sbi, _HW - _W), jnp.bfloat16)
        p_ref[0:_C, :] = dn * myb_ref[0]
        p_ref[_C:2 * _C, :] = sb
        p_ref[2 * _C:3 * _C, :] = up * myb_ref[1]
        z = jnp.dot(w_ref[...], p_ref[...], preferred_element_type=jnp.float32)
        return (_roll(z[0:_C], 1) * mxf_ref[0] + z[_C:2 * _C]
                + _roll(z[2 * _C:3 * _C], _HW - 1) * mxf_ref[1])

    for i in range(_IPS):
        x = x_ref[i]
        h_ref[...] = jnp.maximum(conv(x, w1_ref), 0.0)
        o_ref[i] = jnp.maximum(conv(h_ref[...], w2_ref) + x, 0.0)


def _prep_weights(w_oihw, gamma, beta, mean, var, eps):
    inv = gamma / jnp.sqrt(var + eps)
    wm = jnp.transpose(w_oihw, (3, 0, 2, 1)).reshape(3, _C, 3 * _C)
    wm = wm * inv[None, :, None]
    bias = (beta - mean * inv)[:, None]
    ext = jnp.zeros((3, _C, 8), jnp.float32).at[1, :, 0:1].set(bias)
    return jnp.concatenate([wm, ext], axis=2).reshape(3 * _C, _KW).astype(
        jnp.bfloat16)


def _masks():
    lane = jnp.arange(_HW, dtype=jnp.int32)
    yc, xc = lane // _W, lane % _W
    my = jnp.stack([yc >= 1, yc < _H - 1])
    mx = jnp.stack([xc >= 1, xc < _W - 1])
    myb = my[:, None, :].astype(jnp.bfloat16)
    mxf = mx[:, None, :].astype(jnp.float32)
    return myb, mxf


def kernel(x, w1, gamma1, beta1, mean1, var1, w2, gamma2, beta2, mean2, var2,
           eps=1e-5):
    N = x.shape[0]
    xr = x.reshape(N, _C, _HW).astype(jnp.float32)
    w1m = _prep_weights(w1, gamma1, beta1, mean1, var1, eps)
    w2m = _prep_weights(w2, gamma2, beta2, mean2, var2, eps)
    myb, mxf = _masks()
    bias_rows = jnp.zeros((8, _HW), jnp.bfloat16).at[0, :].set(1.0)

    flops = 4 * N * _HW * 9 * _C * _C
    bytes_accessed = 2 * N * _C * _HW * 4 + 2 * 3 * _C * _KW * 2

    out = pl.pallas_call(
        _bb_kernel,
        out_shape=jax.ShapeDtypeStruct((N, _C, _HW), jnp.float32),
        grid=(N // _IPS,),
        in_specs=[
            pl.BlockSpec((_IPS, _C, _HW), lambda n: (n, 0, 0)),
            pl.BlockSpec((3 * _C, _KW), lambda n: (0, 0)),
            pl.BlockSpec((3 * _C, _KW), lambda n: (0, 0)),
            pl.BlockSpec((2, 1, _HW), lambda n: (0, 0, 0)),
            pl.BlockSpec((2, 1, _HW), lambda n: (0, 0, 0)),
            pl.BlockSpec((8, _HW), lambda n: (0, 0)),
        ],
        out_specs=pl.BlockSpec((_IPS, _C, _HW), lambda n: (n, 0, 0)),
        scratch_shapes=[
            pltpu.VMEM((_KW, _HW), jnp.bfloat16),
            pltpu.VMEM((_C, _HW), jnp.float32),
        ],
        compiler_params=pltpu.CompilerParams(
            dimension_semantics=("parallel",),
            vmem_limit_bytes=100 * 1024 * 1024,
        ),
        cost_estimate=pl.CostEstimate(flops=flops, transcendentals=0,
                                      bytes_accessed=bytes_accessed),
    )(xr, w1m, w2m, myb, mxf, bias_rows)

    return out.reshape(N, _C, _H, _W)
